# baseline (speedup 1.0000x reference)
import functools

import jax
import jax.numpy as jnp
from jax import lax
from jax.experimental import pallas as pl
from jax.experimental.pallas import tpu as pltpu
from jax.experimental.pallas import tpu_sc as plsc

B, F, L, V, D = 1024, 26, 20, 100000, 16
NC, NS = 2, 16
NW = NC * NS
ROWS = F * B * L
RPW = ROWS // NW
IDXW = 128
IRPW = RPW // IDXW
BAGS_PW = RPW // L
CB = 64
CROWS = CB * L
NCHUNK = BAGS_PW // CB
GPC = CROWS // IDXW
CPF = B * L // CROWS

_mesh = plsc.VectorSubcoreMesh(
    core_axis_name="c", subcore_axis_name="s", num_cores=NC, num_subcores=NS
)


@functools.partial(
    pl.kernel,
    out_type=(
        jax.ShapeDtypeStruct((D, ROWS), jnp.float32),
        jax.ShapeDtypeStruct((F * D, B), jnp.float32),
    ),
    mesh=_mesh,
    compiler_params=pltpu.CompilerParams(
        use_tc_tiling_on_sc=False, needs_layout_passes=False),
    scratch_types=(
        pltpu.VMEM((IRPW, IDXW), jnp.int32),
        pltpu.VMEM((CROWS, D), jnp.float32),
        pltpu.VMEM((CROWS, D), jnp.float32),
        pltpu.VMEM((D, CROWS), jnp.float32),
        pltpu.VMEM((CB, D), jnp.float32),
        pltpu.VMEM((D, CB), jnp.float32),
        pltpu.SemaphoreType.DMA,
        pltpu.SemaphoreType.DMA,
    ),
)
def _sparse_arch_sc(idx_hbm, ebc_t, ec_t, ec_out, ebc_out,
                    idx_v, ebc_buf, ec_buf, ecT_v, pooled_v, pooledT_v,
                    sem_e, sem_c):
    wid = lax.axis_index("s") * NC + lax.axis_index("c")
    pltpu.sync_copy(idx_hbm.at[wid], idx_v)
    lane = lax.iota(jnp.int32, 16)

    def chunk_body(c, carry):
        g0 = wid * NCHUNK + c
        f = g0 // CPF
        b0 = (g0 % CPF) * CB
        ebc_dmas = []
        ec_dmas = []
        for j in range(GPC):
            r = c * GPC + j
            ebc_dmas.append(pltpu.async_copy(
                ebc_t.at[f].at[idx_v.at[r]], ebc_buf.at[pl.ds(j * IDXW, IDXW)], sem_e))
            ec_dmas.append(pltpu.async_copy(
                ec_t.at[f].at[idx_v.at[r]], ec_buf.at[pl.ds(j * IDXW, IDXW)], sem_c))
        for dma in ec_dmas:
            dma.wait()

        def ec_t_body(i0, carry2):
            rows = i0 * 16 + lane
            for d in range(D):
                vals = plsc.load_gather(ec_buf, [rows, jnp.full((16,), d, jnp.int32)])
                ecT_v[d, pl.ds(i0 * 16, 16)] = vals
            return carry2

        lax.fori_loop(0, CROWS // 16, ec_t_body, 0, unroll=False)
        pltpu.sync_copy(ecT_v, ec_out.at[:, pl.ds(wid * RPW + c * CROWS, CROWS)])

        for dma in ebc_dmas:
            dma.wait()

        def bag_body(jb, carry2):
            base = jb * L
            acc = ebc_buf[base]
            for l in range(1, L):
                acc = acc + ebc_buf[base + l]
            pooled_v[jb] = acc
            return carry2

        lax.fori_loop(0, CB, bag_body, 0, unroll=False)

        def pl_t_body(i0, carry2):
            rows = i0 * 16 + lane
            for d in range(D):
                vals = plsc.load_gather(pooled_v, [rows, jnp.full((16,), d, jnp.int32)])
                pooledT_v[d, pl.ds(i0 * 16, 16)] = vals
            return carry2

        lax.fori_loop(0, CB // 16, pl_t_body, 0, unroll=False)
        pltpu.sync_copy(pooledT_v, ebc_out.at[pl.ds(f * D, D), pl.ds(b0, CB)])
        return carry

    lax.fori_loop(0, NCHUNK, chunk_body, 0, unroll=False)


def kernel(indices, ebc_tables, ec_tables):
    idx_fm = jnp.transpose(indices, (1, 0, 2)).astype(jnp.int32)
    idx_fm = idx_fm.reshape(NW, IRPW, IDXW)
    ecT, pooledT = _sparse_arch_sc(idx_fm, ebc_tables, ec_tables)
    ebc_values = jnp.transpose(pooledT)
    ec_values = jnp.transpose(ecT)
    return ebc_values, ec_values


# speedup vs baseline: 1.0189x; 1.0189x over previous
import functools

import jax
import jax.numpy as jnp
from jax import lax
from jax.experimental import pallas as pl
from jax.experimental.pallas import tpu as pltpu
from jax.experimental.pallas import tpu_sc as plsc

B, F, L, V, D = 1024, 26, 20, 100000, 16
NC, NS = 2, 16
NW = NC * NS
ROWS = F * B * L
RPW = ROWS // NW
IDXW = 128
IRPW = RPW // IDXW
BAGS_PW = RPW // L
CB = 64
CROWS = CB * L
NCHUNK = BAGS_PW // CB
GPC = CROWS // IDXW
CPF = B * L // CROWS

_mesh = plsc.VectorSubcoreMesh(
    core_axis_name="c", subcore_axis_name="s", num_cores=NC, num_subcores=NS
)


@functools.partial(
    pl.kernel,
    out_type=(
        jax.ShapeDtypeStruct((D, ROWS), jnp.float32),
        jax.ShapeDtypeStruct((F * D, B), jnp.float32),
    ),
    mesh=_mesh,
    compiler_params=pltpu.CompilerParams(
        use_tc_tiling_on_sc=False, needs_layout_passes=False),
    scratch_types=(
        pltpu.VMEM((IRPW, IDXW), jnp.int32),
        pltpu.VMEM((CROWS, D), jnp.float32),
        pltpu.VMEM((CROWS, D), jnp.float32),
        pltpu.VMEM((D, CROWS), jnp.float32),
        pltpu.VMEM((CB, D), jnp.float32),
        pltpu.VMEM((D, CB), jnp.float32),
        pltpu.SemaphoreType.DMA,
        pltpu.SemaphoreType.DMA,
    ),
)
def _sparse_arch_sc(idx_hbm, ebc_t, ec_t, ec_out, ebc_out,
                    idx_v, ebc_buf, ec_buf, ecT_v, pooled_v, pooledT_v,
                    sem_e, sem_c):
    wid = lax.axis_index("s") * NC + lax.axis_index("c")
    pltpu.sync_copy(idx_hbm.at[wid], idx_v)
    lane = lax.iota(jnp.int32, 16)
    diag_cols = [jnp.bitwise_and(lane + dd, 15) for dd in range(D)]

    def transpose16(src, dst, r0):
        rows = r0 + lane
        for dd in range(D):
            vals = plsc.load_gather(src, [rows, diag_cols[dd]])
            plsc.store_scatter(dst, [diag_cols[dd], rows], vals)

    def chunk_body(c, carry):
        g0 = wid * NCHUNK + c
        f = g0 // CPF
        b0 = (g0 % CPF) * CB
        ebc_dmas = []
        ec_dmas = []
        for j in range(GPC):
            r = c * GPC + j
            ebc_dmas.append(pltpu.async_copy(
                ebc_t.at[f].at[idx_v.at[r]], ebc_buf.at[pl.ds(j * IDXW, IDXW)], sem_e))
            ec_dmas.append(pltpu.async_copy(
                ec_t.at[f].at[idx_v.at[r]], ec_buf.at[pl.ds(j * IDXW, IDXW)], sem_c))
        for dma in ec_dmas:
            dma.wait()

        def ec_t_body(i0, carry2):
            transpose16(ec_buf, ecT_v, i0 * 16)
            return carry2

        lax.fori_loop(0, CROWS // 16, ec_t_body, 0, unroll=False)
        pltpu.sync_copy(ecT_v, ec_out.at[:, pl.ds(wid * RPW + c * CROWS, CROWS)])

        for dma in ebc_dmas:
            dma.wait()

        def bag_body(jb, carry2):
            base = jb * L
            acc = ebc_buf[base]
            for l in range(1, L):
                acc = acc + ebc_buf[base + l]
            pooled_v[jb] = acc
            return carry2

        lax.fori_loop(0, CB, bag_body, 0, unroll=False)

        for i0 in range(CB // 16):
            transpose16(pooled_v, pooledT_v, i0 * 16)
        pltpu.sync_copy(pooledT_v, ebc_out.at[pl.ds(f * D, D), pl.ds(b0, CB)])
        return carry

    lax.fori_loop(0, NCHUNK, chunk_body, 0, unroll=False)


def kernel(indices, ebc_tables, ec_tables):
    idx_fm = jnp.transpose(indices, (1, 0, 2)).astype(jnp.int32)
    idx_fm = idx_fm.reshape(NW, IRPW, IDXW)
    ecT, pooledT = _sparse_arch_sc(idx_fm, ebc_tables, ec_tables)
    ebc_values = jnp.transpose(pooledT)
    ec_values = jnp.transpose(ecT)
    return ebc_values, ec_values


# speedup vs baseline: 1.1315x; 1.1105x over previous
import functools

import jax
import jax.numpy as jnp
from jax import lax
from jax.experimental import pallas as pl
from jax.experimental.pallas import tpu as pltpu
from jax.experimental.pallas import tpu_sc as plsc

B, F, L, V, D = 1024, 26, 20, 100000, 16
NC, NS = 2, 16
NW = NC * NS
ROWS = F * B * L
RPW = ROWS // NW
BAGS_PW = RPW // L
CB = 64
CROWS = CB * L
CELEM = CROWS * D
NCHUNK = BAGS_PW // CB
CPF = B * L // CROWS

_mesh = plsc.VectorSubcoreMesh(
    core_axis_name="c", subcore_axis_name="s", num_cores=NC, num_subcores=NS
)


@functools.partial(
    pl.kernel,
    out_type=(
        jax.ShapeDtypeStruct((D, ROWS), jnp.float32),
        jax.ShapeDtypeStruct((F * D, B), jnp.float32),
    ),
    mesh=_mesh,
    compiler_params=pltpu.CompilerParams(
        use_tc_tiling_on_sc=False, needs_layout_passes=False),
    scratch_types=(
        pltpu.VMEM((CELEM,), jnp.int32),
        pltpu.VMEM((CELEM,), jnp.int32),
        pltpu.VMEM((CELEM,), jnp.float32),
        pltpu.VMEM((CELEM,), jnp.float32),
        pltpu.VMEM((CB, D), jnp.float32),
        pltpu.VMEM((D, CB), jnp.float32),
        pltpu.SemaphoreType.DMA,
        pltpu.SemaphoreType.DMA,
        pltpu.SemaphoreType.DMA,
    ),
)
def _sparse_arch_sc(erm_hbm, edm_hbm, ebc_t, ec_t, ec_out, ebc_out,
                    erm_v, edm_v, ebc_buf, ec_buf, pooled_v, pooledT_v,
                    sem_i, sem_e, sem_c):
    wid = lax.axis_index("s") * NC + lax.axis_index("c")
    lane = lax.iota(jnp.int32, 16)
    diag_cols = [jnp.bitwise_and(lane + dd, 15) for dd in range(D)]

    def chunk_body(c, carry):
        g0 = wid * NCHUNK + c
        f = g0 // CPF
        b0 = (g0 % CPF) * CB
        i_rm = pltpu.async_copy(erm_hbm.at[pl.ds(g0 * CELEM, CELEM)], erm_v, sem_i)
        i_dm = pltpu.async_copy(edm_hbm.at[pl.ds(g0 * CELEM, CELEM)], edm_v, sem_i)
        i_rm.wait()
        g_e = pltpu.async_copy(ebc_t.at[erm_v], ebc_buf, sem_e)
        i_dm.wait()
        g_c = pltpu.async_copy(ec_t.at[edm_v], ec_buf, sem_c)
        g_c.wait()
        col0 = wid * RPW + c * CROWS
        ec_wr = [pltpu.async_copy(ec_buf.at[pl.ds(d * CROWS, CROWS)],
                                  ec_out.at[d, pl.ds(col0, CROWS)], sem_c)
                 for d in range(D)]
        g_e.wait()

        def bag_body(jb, carry2):
            base = jb * L * D
            acc = ebc_buf[pl.ds(base, D)]
            for l in range(1, L):
                acc = acc + ebc_buf[pl.ds(base + l * D, D)]
            pooled_v[jb] = acc
            return carry2

        lax.fori_loop(0, CB, bag_body, 0, unroll=False)
        for i0 in range(CB // 16):
            rows = i0 * 16 + lane
            for dd in range(D):
                vals = plsc.load_gather(pooled_v, [rows, diag_cols[dd]])
                plsc.store_scatter(pooledT_v, [diag_cols[dd], rows], vals)
        pltpu.sync_copy(pooledT_v, ebc_out.at[pl.ds(f * D, D), pl.ds(b0, CB)])
        for dma in ec_wr:
            dma.wait()
        return carry

    lax.fori_loop(0, NCHUNK, chunk_body, 0, unroll=False)


def kernel(indices, ebc_tables, ec_tables):
    v_fm = jnp.transpose(indices, (1, 0, 2)).astype(jnp.int32)
    v_fm = v_fm.reshape(NW, NCHUNK, CROWS)
    offs = jnp.arange(F * D, dtype=jnp.int32).reshape(F, D) * V
    f_of_chunk = (jnp.arange(NW * NCHUNK, dtype=jnp.int32) // CPF).reshape(NW, NCHUNK)
    o_c = offs[f_of_chunk]
    e_rm = (jnp.repeat(v_fm, D, axis=-1) + jnp.tile(o_c, (1, 1, CROWS))).reshape(-1)
    e_dm = (jnp.tile(v_fm, (1, 1, D)) + jnp.repeat(o_c, CROWS, axis=-1)).reshape(-1)
    ebc_flat = jnp.transpose(ebc_tables, (0, 2, 1)).reshape(F * D * V)
    ec_flat = jnp.transpose(ec_tables, (0, 2, 1)).reshape(F * D * V)
    ecT, pooledT = _sparse_arch_sc(e_rm, e_dm, ebc_flat, ec_flat)
    return jnp.transpose(pooledT), jnp.transpose(ecT)


# speedup vs baseline: 1.3390x; 1.1834x over previous
import functools

import jax
import jax.numpy as jnp
from jax import lax
from jax.experimental import pallas as pl
from jax.experimental.pallas import tpu as pltpu
from jax.experimental.pallas import tpu_sc as plsc

B, F, L, V, D = 1024, 26, 20, 100000, 16
NC, NS = 2, 16
NW = NC * NS
ROWS = F * B * L
RPW = ROWS // NW
BAGS_PW = RPW // L
CB = 64
CROWS = CB * L
CELEM = CROWS * D
NCHUNK = BAGS_PW // CB
CPF = B * L // CROWS
NPH = CROWS // 80

_mesh = plsc.VectorSubcoreMesh(
    core_axis_name="c", subcore_axis_name="s", num_cores=NC, num_subcores=NS
)


@functools.partial(
    pl.kernel,
    out_type=(
        jax.ShapeDtypeStruct((D, ROWS), jnp.float32),
        jax.ShapeDtypeStruct((F * D, B), jnp.float32),
    ),
    mesh=_mesh,
    compiler_params=pltpu.CompilerParams(
        use_tc_tiling_on_sc=False, needs_layout_passes=False),
    scratch_types=(
        pltpu.VMEM((CELEM,), jnp.int32),
        pltpu.VMEM((CELEM,), jnp.float32),
        pltpu.VMEM((CELEM,), jnp.float32),
        pltpu.VMEM((D, CB), jnp.float32),
        pltpu.SemaphoreType.DMA,
        pltpu.SemaphoreType.DMA,
        pltpu.SemaphoreType.DMA,
    ),
)
def _sparse_arch_sc(edm_hbm, ebc_t, ec_t, ec_out, ebc_out,
                    edm_v, ebc_buf, ec_buf, pooled_v, sem_i, sem_e, sem_c):
    wid = lax.axis_index("s") * NC + lax.axis_index("c")
    lane = lax.iota(jnp.int32, 16)
    zeros = jnp.zeros((16,), jnp.float32)
    bag_pat = [(p * 16 + lane) // 20 for p in range(5)]

    def chunk_body(c, carry):
        g0 = wid * NCHUNK + c
        f = g0 // CPF
        b0 = (g0 % CPF) * CB
        pltpu.async_copy(edm_hbm.at[pl.ds(g0 * CELEM, CELEM)], edm_v, sem_i).wait()
        g_e = pltpu.async_copy(ebc_t.at[edm_v], ebc_buf, sem_e)
        g_c = pltpu.async_copy(ec_t.at[edm_v], ec_buf, sem_c)
        g_c.wait()
        col0 = wid * RPW + c * CROWS
        ec_wr = [pltpu.async_copy(ec_buf.at[pl.ds(d * CROWS, CROWS)],
                                  ec_out.at[d, pl.ds(col0, CROWS)], sem_c)
                 for d in range(D)]
        g_e.wait()
        for d in range(D):
            for i0 in range(CB // 16):
                pooled_v[d, pl.ds(i0 * 16, 16)] = zeros

        def pool_body(ph, carry2):
            jb0 = ph * 4
            for d in range(D):
                base = d * CROWS + ph * 80
                for p in range(5):
                    vals = ebc_buf[pl.ds(base + p * 16, 16)]
                    plsc.addupdate_scatter(
                        pooled_v, [jnp.full((16,), d, jnp.int32), jb0 + bag_pat[p]],
                        vals)
            return carry2

        lax.fori_loop(0, NPH, pool_body, 0, unroll=False)
        pltpu.sync_copy(pooled_v, ebc_out.at[pl.ds(f * D, D), pl.ds(b0, CB)])
        for dma in ec_wr:
            dma.wait()
        return carry

    lax.fori_loop(0, NCHUNK, chunk_body, 0, unroll=False)


def kernel(indices, ebc_tables, ec_tables):
    v_fm = jnp.transpose(indices, (1, 0, 2)).astype(jnp.int32)
    v_fm = v_fm.reshape(NW, NCHUNK, CROWS)
    offs = jnp.arange(F * D, dtype=jnp.int32).reshape(F, D) * V
    f_of_chunk = (jnp.arange(NW * NCHUNK, dtype=jnp.int32) // CPF).reshape(NW, NCHUNK)
    o_c = offs[f_of_chunk]
    e_dm = (v_fm[:, :, None, :] + o_c[:, :, :, None]).reshape(-1)
    ebc_flat = jnp.transpose(ebc_tables, (0, 2, 1)).reshape(F * D * V)
    ec_flat = jnp.transpose(ec_tables, (0, 2, 1)).reshape(F * D * V)
    ecT, pooledT = _sparse_arch_sc(e_dm, ebc_flat, ec_flat)
    return jnp.transpose(pooledT), jnp.transpose(ecT)


# speedup vs baseline: 1.4755x; 1.1019x over previous
import functools

import jax
import jax.numpy as jnp
from jax import lax
from jax.experimental import pallas as pl
from jax.experimental.pallas import tpu as pltpu
from jax.experimental.pallas import tpu_sc as plsc

B, F, L, V, D = 1024, 26, 20, 100000, 16
NC, NS = 2, 16
NW = NC * NS
ROWS = F * B * L
RPW = ROWS // NW
BAGS_PW = RPW // L
CB = 64
CROWS = CB * L
CELEM = CROWS * D
NCHUNK = BAGS_PW // CB
CPF = B * L // CROWS
NPH = CROWS // 80

_mesh = plsc.VectorSubcoreMesh(
    core_axis_name="c", subcore_axis_name="s", num_cores=NC, num_subcores=NS
)


@functools.partial(
    pl.kernel,
    out_type=(
        jax.ShapeDtypeStruct((D, ROWS), jnp.float32),
        jax.ShapeDtypeStruct((F * D, B), jnp.float32),
    ),
    mesh=_mesh,
    compiler_params=pltpu.CompilerParams(
        use_tc_tiling_on_sc=False, needs_layout_passes=False),
    scratch_types=(
        pltpu.VMEM((2, CELEM), jnp.int32),
        pltpu.VMEM((2, CELEM), jnp.float32),
        pltpu.VMEM((2, CELEM), jnp.float32),
        pltpu.VMEM((D, CB), jnp.float32),
        pltpu.SemaphoreType.DMA,
        pltpu.SemaphoreType.DMA,
        pltpu.SemaphoreType.DMA,
        pltpu.SemaphoreType.DMA,
        pltpu.SemaphoreType.DMA,
        pltpu.SemaphoreType.DMA,
    ),
)
def _sparse_arch_sc(edm_hbm, ebc_t, ec_t, ec_out, ebc_out,
                    edm_v, ebc_buf, ec_buf, pooled_v,
                    sem_i0, sem_i1, sem_e0, sem_e1, sem_c0, sem_c1):
    wid = lax.axis_index("s") * NC + lax.axis_index("c")
    lane = lax.iota(jnp.int32, 16)
    zeros = jnp.zeros((16,), jnp.float32)
    bag_pat = [(p * 16 + lane) // 20 for p in range(5)]
    sem_i = [sem_i0, sem_i1]
    sem_e = [sem_e0, sem_e1]
    sem_c = [sem_c0, sem_c1]

    def start_idx(c):
        s = c % 2
        g0 = wid * NCHUNK + c
        return pltpu.async_copy(
            edm_hbm.at[pl.ds(g0 * CELEM, CELEM)], edm_v.at[s], sem_i[s])

    def start_gathers(c):
        s = c % 2
        return (pltpu.async_copy(ebc_t.at[edm_v.at[s]], ebc_buf.at[s], sem_e[s]),
                pltpu.async_copy(ec_t.at[edm_v.at[s]], ec_buf.at[s], sem_c[s]))

    idx_dma = start_idx(0)
    idx_dma.wait()
    gathers = start_gathers(0)
    idx_next = start_idx(1) if NCHUNK > 1 else None

    for c in range(NCHUNK):
        s = c % 2
        g0 = wid * NCHUNK + c
        f = g0 // CPF
        b0 = (g0 % CPF) * CB
        g_e, g_c = gathers
        g_c.wait()
        col0 = wid * RPW + c * CROWS
        ec_wr = [pltpu.async_copy(ec_buf.at[s].at[pl.ds(d * CROWS, CROWS)],
                                  ec_out.at[d, pl.ds(col0, CROWS)], sem_c[s])
                 for d in range(D)]
        g_e.wait()
        if c + 1 < NCHUNK:
            idx_next.wait()
            gathers = start_gathers(c + 1)
            if c + 2 < NCHUNK:
                idx_next = start_idx(c + 2)
        for d in range(D):
            for i0 in range(CB // 16):
                pooled_v[d, pl.ds(i0 * 16, 16)] = zeros

        def pool_body(ph, carry2, _s=s):
            jb0 = ph * 4
            for d in range(D):
                base = d * CROWS + ph * 80
                for p in range(5):
                    vals = ebc_buf[_s, pl.ds(base + p * 16, 16)]
                    plsc.addupdate_scatter(
                        pooled_v, [jnp.full((16,), d, jnp.int32), jb0 + bag_pat[p]],
                        vals)
            return carry2

        lax.fori_loop(0, NPH, pool_body, 0, unroll=False)
        pltpu.sync_copy(pooled_v, ebc_out.at[pl.ds(f * D, D), pl.ds(b0, CB)])
        for dma in ec_wr:
            dma.wait()


def kernel(indices, ebc_tables, ec_tables):
    v_fm = jnp.transpose(indices, (1, 0, 2)).astype(jnp.int32)
    v_fm = v_fm.reshape(NW, NCHUNK, CROWS)
    offs = jnp.arange(F * D, dtype=jnp.int32).reshape(F, D) * V
    f_of_chunk = (jnp.arange(NW * NCHUNK, dtype=jnp.int32) // CPF).reshape(NW, NCHUNK)
    o_c = offs[f_of_chunk]
    e_dm = (v_fm[:, :, None, :] + o_c[:, :, :, None]).reshape(-1)
    ebc_flat = jnp.transpose(ebc_tables, (0, 2, 1)).reshape(F * D * V)
    ec_flat = jnp.transpose(ec_tables, (0, 2, 1)).reshape(F * D * V)
    ecT, pooledT = _sparse_arch_sc(e_dm, ebc_flat, ec_flat)
    return jnp.transpose(pooledT), jnp.transpose(ecT)
